# Initial kernel scaffold; baseline (speedup 1.0000x reference)
import jax
import jax.numpy as jnp
from jax.experimental import pallas as pl


def kernel(feature, edge_index, W, b):
    raise NotImplementedError("write your pallas kernel here")


# speedup vs baseline: 5.3407x; 5.3407x over previous
import functools

import jax
import jax.numpy as jnp
from jax import lax
from jax.experimental import pallas as pl
from jax.experimental.pallas import tpu as pltpu
from jax.experimental.pallas import tpu_sc as plsc

N_NODES = 10000
N_EDGES = 160000
D_HALF = 128

NC = 2
NS = 16
EDGES_PER_SUBCORE = N_EDGES // NS
CHUNK = 80
NCHUNKS = EDGES_PER_SUBCORE // CHUNK
WB_ROWS = 80
WB_CHUNKS = N_NODES // WB_ROWS
WB_ITERS = (WB_CHUNKS + NS - 1) // NS


def _sc_gather_scatter(flo, fhi, src, dst):
    mesh = plsc.VectorSubcoreMesh(
        core_axis_name="c", subcore_axis_name="s",
        num_cores=NC, num_subcores=NS)

    @functools.partial(
        pl.kernel,
        out_type=jax.ShapeDtypeStruct((NC * N_NODES, D_HALF), jnp.float32),
        mesh=mesh,
        scratch_types=[
            pltpu.VMEM_SHARED((N_NODES, D_HALF), jnp.float32),
            pltpu.VMEM((NCHUNKS, CHUNK), jnp.int32),
            pltpu.VMEM((NCHUNKS, CHUNK), jnp.int32),
            pltpu.VMEM((CHUNK, D_HALF), jnp.float32),
            pltpu.SemaphoreType.DMA,
        ],
    )
    def k(flo_hbm, fhi_hbm, src_hbm, dst_hbm, out_hbm,
          acc, src_v, dst_v, rows_v, sem):
        c = lax.axis_index("c")
        s = lax.axis_index("s")

        zv = jnp.zeros((16,), jnp.float32)

        def zrow(i, carry):
            for jj in range(D_HALF // 16):
                rows_v[i, pl.ds(jj * 16, 16)] = zv
            return carry

        lax.fori_loop(0, WB_ROWS, zrow, 0)
        for i in range(WB_ITERS):
            idx = s + i * NS

            @pl.when(idx < WB_CHUNKS)
            def _():
                pltpu.sync_copy(rows_v, acc.at[pl.ds(idx * WB_ROWS, WB_ROWS)])

        pltpu.sync_copy(src_hbm.at[s], src_v)
        pltpu.sync_copy(dst_hbm.at[s], dst_v)

        plsc.subcore_barrier()

        def do_edges(feat_hbm):
            def step(j, carry):
                pltpu.async_copy(feat_hbm.at[src_v.at[j]], rows_v, sem).wait()
                pltpu.sync_copy(rows_v, acc.at[dst_v.at[j]], add=True)
                return carry
            lax.fori_loop(0, NCHUNKS, step, 0)

        @pl.when(c == 0)
        def _():
            do_edges(flo_hbm)

        @pl.when(c == 1)
        def _():
            do_edges(fhi_hbm)

        plsc.subcore_barrier()

        for i in range(WB_ITERS):
            idx = s + i * NS

            @pl.when(idx < WB_CHUNKS)
            def _():
                off = idx * WB_ROWS
                pltpu.sync_copy(acc.at[pl.ds(off, WB_ROWS)], rows_v)
                pltpu.sync_copy(rows_v, out_hbm.at[pl.ds(c * N_NODES + off, WB_ROWS)])

    return k(flo, fhi, src, dst)


def _tc_body(x_ref, wt_ref, b_ref, o_ref):
    acc = jnp.dot(x_ref[0], wt_ref[:D_HALF, :],
                  preferred_element_type=jnp.float32)
    acc += jnp.dot(x_ref[1], wt_ref[D_HALF:, :],
                   preferred_element_type=jnp.float32)
    o_ref[...] = jnp.maximum(acc + b_ref[...], 0.0)


def _tc_linear_relu(agg2, wt, b2):
    blk = 2000
    grid = N_NODES // blk
    return pl.pallas_call(
        _tc_body,
        grid=(grid,),
        in_specs=[
            pl.BlockSpec((2, blk, D_HALF), lambda i: (0, i, 0)),
            pl.BlockSpec((2 * D_HALF, 2 * D_HALF), lambda i: (0, 0)),
            pl.BlockSpec((1, 2 * D_HALF), lambda i: (0, 0)),
        ],
        out_specs=pl.BlockSpec((blk, 2 * D_HALF), lambda i: (i, 0)),
        out_shape=jax.ShapeDtypeStruct((N_NODES, 2 * D_HALF), jnp.float32),
    )(agg2, wt, b2)


def kernel(feature, edge_index, W, b):
    src = edge_index[0].astype(jnp.int32).reshape(NS, NCHUNKS, CHUNK)
    dst = edge_index[1].astype(jnp.int32).reshape(NS, NCHUNKS, CHUNK)
    flo = feature[:, :D_HALF]
    fhi = feature[:, D_HALF:]
    agg2 = _sc_gather_scatter(flo, fhi, src, dst)
    return _tc_linear_relu(agg2.reshape(NC, N_NODES, D_HALF), W.T,
                           b.reshape(1, 2 * D_HALF))
